# baseline (speedup 1.0000x reference)
import functools

import jax
import jax.numpy as jnp
from jax import lax
from jax.experimental import pallas as pl
from jax.experimental.pallas import tpu as pltpu
from jax.experimental.pallas import tpu_sc as plsc

_B = 256
_N_PV = 1400
_F = 64
_EMB = 32
_OUTC = 2 * _F + _F + _EMB + 2
_NROWS = _B * _N_PV

_NC = 2
_NS = 16
_NW = _NC * _NS
_BPW = _B // _NW
_R = 200
_NCH = _N_PV // _R


def _sc_build(y2, x2, idx, t4, azel, table):
    mesh = plsc.VectorSubcoreMesh(core_axis_name="c", subcore_axis_name="s")

    @functools.partial(
        pl.kernel,
        mesh=mesh,
        compiler_params=pltpu.CompilerParams(use_tc_tiling_on_sc=False),
        out_type=jax.ShapeDtypeStruct((_NROWS, _OUTC), jnp.float32),
        scratch_types=[
            pltpu.VMEM((_R,), jnp.int32),
            pltpu.VMEM((_R, _F), jnp.float32),
            pltpu.VMEM((_R, _F), jnp.float32),
            pltpu.VMEM((_R, _EMB), jnp.float32),
            pltpu.VMEM((_R, _F), jnp.float32),
            pltpu.VMEM((4, 16), jnp.float32),
            pltpu.VMEM((_R, 2), jnp.float32),
            pltpu.SemaphoreType.DMA,
        ],
    )
    def k(y_hbm, x_hbm, idx_hbm, t_hbm, azel_hbm, table_hbm, out_hbm,
          idx_v, y_v, x_v, emb_v, t_stage, t_buf, azel_stage, sem):
        wid = lax.axis_index("s") * _NC + lax.axis_index("c")

        def batch_body(kb, carry):
            b = wid * _BPW + kb
            pltpu.sync_copy(t_hbm.at[b], t_buf)
            pltpu.sync_copy(azel_hbm.at[b], azel_stage)
            t0 = t_buf[0]
            t1 = t_buf[1]
            t2 = t_buf[2]
            t3 = t_buf[3]

            def fill_body(r, c2):
                t_stage[r, pl.ds(0, 16)] = t0
                t_stage[r, pl.ds(16, 16)] = t1
                t_stage[r, pl.ds(32, 16)] = t2
                t_stage[r, pl.ds(48, 16)] = t3
                return c2

            lax.fori_loop(0, _R, fill_body, 0)

            for c in range(_NCH):
                r0 = b * _N_PV + c * _R
                rows_i = pl.ds(r0, _R)
                pltpu.sync_copy(idx_hbm.at[rows_i], idx_v)
                pltpu.async_copy(table_hbm.at[idx_v], emb_v, sem).wait()
                pltpu.sync_copy(y_hbm.at[rows_i], y_v)
                pltpu.sync_copy(x_hbm.at[rows_i], x_v)
                pltpu.sync_copy(y_v, out_hbm.at[rows_i, pl.ds(0, _F)])
                pltpu.sync_copy(x_v, out_hbm.at[rows_i, pl.ds(_F, _F)])
                pltpu.sync_copy(t_stage, out_hbm.at[rows_i, pl.ds(2 * _F, _F)])
                pltpu.sync_copy(emb_v, out_hbm.at[rows_i, pl.ds(3 * _F, _EMB)])
                pltpu.sync_copy(azel_stage,
                                out_hbm.at[rows_i, pl.ds(3 * _F + _EMB, 2)])
            return carry

        lax.fori_loop(0, _BPW, batch_body, 0)

    return k(y2, x2, idx, t4, azel, table)


def kernel(pv_y_osgb_fourier, pv_x_osgb_fourier, pv_system_row_number,
           pv_time_utc_fourier, pv_x_osgb, solar_azimuth, solar_elevation,
           pv_embedding):
    y2 = pv_y_osgb_fourier.reshape(_NROWS, _F)
    x2 = pv_x_osgb_fourier.reshape(_NROWS, _F)
    idx = pv_system_row_number.reshape(-1).astype(jnp.int32)
    t4 = pv_time_utc_fourier[:, 12].reshape(_B, 4, 16)
    azel = jnp.broadcast_to(
        jnp.stack([solar_azimuth[:, 12], solar_elevation[:, 12]], axis=-1)
        [:, None, :], (_B, _R, 2))
    out2 = _sc_build(y2, x2, idx, t4, azel, pv_embedding)
    return out2.reshape(_B, _N_PV, _OUTC)


# speedup vs baseline: 1.1278x; 1.1278x over previous
import functools

import jax
import jax.numpy as jnp
from jax import lax
from jax.experimental import pallas as pl
from jax.experimental.pallas import tpu as pltpu
from jax.experimental.pallas import tpu_sc as plsc

_B = 256
_N_PV = 1400
_F = 64
_EMB = 32
_OUTC = 2 * _F + _F + _EMB + 2
_NROWS = _B * _N_PV

_NC = 2
_NS = 16
_NW = _NC * _NS
_BPW = _B // _NW
_R = 280
_NCH = _N_PV // _R


def _sc_build(y2, x2, idx, t4, azel, table):
    mesh = plsc.VectorSubcoreMesh(core_axis_name="c", subcore_axis_name="s")

    @functools.partial(
        pl.kernel,
        mesh=mesh,
        compiler_params=pltpu.CompilerParams(use_tc_tiling_on_sc=False),
        out_type=jax.ShapeDtypeStruct((_NROWS, _OUTC), jnp.float32),
        scratch_types=[
            pltpu.VMEM((2, _R), jnp.int32),
            pltpu.VMEM((2, _R, _F), jnp.float32),
            pltpu.VMEM((2, _R, _F), jnp.float32),
            pltpu.VMEM((2, _R, _EMB), jnp.float32),
            pltpu.VMEM((_R, _F), jnp.float32),
            pltpu.VMEM((4, 16), jnp.float32),
            pltpu.VMEM((_R, 2), jnp.float32),
            pltpu.SemaphoreType.DMA((2,)),
            pltpu.SemaphoreType.DMA((2,)),
            pltpu.SemaphoreType.DMA((2,)),
            pltpu.SemaphoreType.DMA((2,)),
        ],
    )
    def k(y_hbm, x_hbm, idx_hbm, t_hbm, azel_hbm, table_hbm, out_hbm,
          idx_v, y_v, x_v, emb_v, t_stage, t_buf, azel_stage,
          sem_idx, sem_in, sem_g, sem_out):
        wid = lax.axis_index("s") * _NC + lax.axis_index("c")

        def start_inputs(b, c, p):
            rows = pl.ds(b * _N_PV + c * _R, _R)
            h_idx = pltpu.async_copy(idx_hbm.at[rows], idx_v.at[p],
                                     sem_idx.at[p])
            h_y = pltpu.async_copy(y_hbm.at[rows], y_v.at[p], sem_in.at[p])
            h_x = pltpu.async_copy(x_hbm.at[rows], x_v.at[p], sem_in.at[p])
            return (h_idx, h_y, h_x)

        def batch_body(kb, carry):
            b = wid * _BPW + kb
            pltpu.sync_copy(t_hbm.at[b], t_buf)
            pltpu.sync_copy(azel_hbm.at[b], azel_stage)
            t0 = t_buf[0]
            t1 = t_buf[1]
            t2 = t_buf[2]
            t3 = t_buf[3]

            def fill_body(r, c2):
                t_stage[r, pl.ds(0, 16)] = t0
                t_stage[r, pl.ds(16, 16)] = t1
                t_stage[r, pl.ds(32, 16)] = t2
                t_stage[r, pl.ds(48, 16)] = t3
                return c2

            lax.fori_loop(0, _R, fill_body, 0)

            pending = {}
            ins = {0: start_inputs(b, 0, 0)}
            for c in range(_NCH):
                p = c & 1
                q = 1 - p
                rows = pl.ds(b * _N_PV + c * _R, _R)
                h_idx, h_y, h_x = ins[p]
                h_idx.wait()
                h_g = pltpu.async_copy(table_hbm.at[idx_v.at[p]],
                                       emb_v.at[p], sem_g.at[p])
                h_y.wait()
                h_x.wait()
                outs = [
                    pltpu.async_copy(y_v.at[p],
                                     out_hbm.at[rows, pl.ds(0, _F)],
                                     sem_out.at[p]),
                    pltpu.async_copy(x_v.at[p],
                                     out_hbm.at[rows, pl.ds(_F, _F)],
                                     sem_out.at[p]),
                    pltpu.async_copy(t_stage,
                                     out_hbm.at[rows, pl.ds(2 * _F, _F)],
                                     sem_out.at[p]),
                    pltpu.async_copy(azel_stage,
                                     out_hbm.at[rows,
                                                pl.ds(3 * _F + _EMB, 2)],
                                     sem_out.at[p]),
                ]
                h_g.wait()
                outs.append(
                    pltpu.async_copy(emb_v.at[p],
                                     out_hbm.at[rows, pl.ds(3 * _F, _EMB)],
                                     sem_out.at[p]))
                pending[p] = outs
                if c + 1 < _NCH:
                    for h in pending.pop(q, ()):
                        h.wait()
                    ins[q] = start_inputs(b, c + 1, q)
            for outs in pending.values():
                for h in outs:
                    h.wait()
            return carry

        lax.fori_loop(0, _BPW, batch_body, 0)

    return k(y2, x2, idx, t4, azel, table)


def kernel(pv_y_osgb_fourier, pv_x_osgb_fourier, pv_system_row_number,
           pv_time_utc_fourier, pv_x_osgb, solar_azimuth, solar_elevation,
           pv_embedding):
    y2 = pv_y_osgb_fourier.reshape(_NROWS, _F)
    x2 = pv_x_osgb_fourier.reshape(_NROWS, _F)
    idx = pv_system_row_number.reshape(-1).astype(jnp.int32)
    t4 = pv_time_utc_fourier[:, 12].reshape(_B, 4, 16)
    azel = jnp.broadcast_to(
        jnp.stack([solar_azimuth[:, 12], solar_elevation[:, 12]], axis=-1)
        [:, None, :], (_B, _R, 2))
    out2 = _sc_build(y2, x2, idx, t4, azel, pv_embedding)
    return out2.reshape(_B, _N_PV, _OUTC)


# speedup vs baseline: 1.6128x; 1.4301x over previous
import functools

import jax
import jax.numpy as jnp
from jax import lax
from jax.experimental import pallas as pl
from jax.experimental.pallas import tpu as pltpu
from jax.experimental.pallas import tpu_sc as plsc

_B = 256
_N_PV = 1400
_F = 64
_EMB = 32
_OUTC = 2 * _F + _F + _EMB + 2
_NROWS = _B * _N_PV

_NC = 2
_NS = 16
_NW = _NC * _NS
_RPW = _NROWS // _NW
_R = 320
_NCH = _RPW // _R

_GB = 4
_GRID = _B // _GB


def _sc_gather(idx, table128):
    mesh = plsc.VectorSubcoreMesh(core_axis_name="c", subcore_axis_name="s")

    @functools.partial(
        pl.kernel,
        mesh=mesh,
        compiler_params=pltpu.CompilerParams(use_tc_tiling_on_sc=False),
        out_type=jax.ShapeDtypeStruct((_NROWS, 128), jnp.float32),
        scratch_types=[
            pltpu.VMEM((2, _R), jnp.int32),
            pltpu.VMEM((2, _R, 128), jnp.float32),
            pltpu.SemaphoreType.DMA((2,)),
            pltpu.SemaphoreType.DMA((2,)),
            pltpu.SemaphoreType.DMA((2,)),
        ],
    )
    def k(idx_hbm, table_hbm, out_hbm, idx_v, emb_v,
          sem_idx, sem_g, sem_out):
        wid = lax.axis_index("s") * _NC + lax.axis_index("c")
        base = wid * _RPW

        def start_idx(c, p):
            return pltpu.async_copy(idx_hbm.at[pl.ds(base + c * _R, _R)],
                                    idx_v.at[p], sem_idx.at[p])

        ins = {0: start_idx(0, 0)}
        outs = {}
        for c in range(_NCH):
            p = c & 1
            q = 1 - p
            if c + 1 < _NCH:
                ins[q] = start_idx(c + 1, q)
            ins.pop(p).wait()
            if c >= 2:
                outs.pop(p).wait()
            pltpu.async_copy(table_hbm.at[idx_v.at[p]], emb_v.at[p],
                             sem_g.at[p]).wait()
            outs[p] = pltpu.async_copy(
                emb_v.at[p], out_hbm.at[pl.ds(base + c * _R, _R)],
                sem_out.at[p])
        for h in outs.values():
            h.wait()

    return k(idx, table128)


def _fix(v):
    return jnp.nan_to_num(v)


def _tc_body(y_ref, x_ref, emb_ref, t_ref, az_ref, el_ref, out_ref):
    g = pl.program_id(0)
    for bb in range(_GB):
        y = _fix(y_ref[bb])
        x = _fix(x_ref[bb])
        e = _fix(emb_ref[pl.ds(bb * _N_PV, _N_PV), pl.ds(0, _EMB)])
        t = _fix(t_ref[0, bb])
        az = _fix(az_ref[g * _GB + bb])
        el = _fix(el_ref[g * _GB + bb])
        tb = jnp.broadcast_to(t[None, :], (_N_PV, _F))
        azc = jnp.full((_N_PV, 1), az, jnp.float32)
        elc = jnp.full((_N_PV, 1), el, jnp.float32)
        out_ref[bb] = jnp.concatenate([y, x, tb, e, azc, elc], axis=-1)


def _tc_assemble(y, x, emb, t, az, el):
    return pl.pallas_call(
        _tc_body,
        grid=(_GRID,),
        in_specs=[
            pl.BlockSpec((_GB, _N_PV, _F), lambda i: (i, 0, 0)),
            pl.BlockSpec((_GB, _N_PV, _F), lambda i: (i, 0, 0)),
            pl.BlockSpec((_GB * _N_PV, 128), lambda i: (i, 0)),
            pl.BlockSpec((1, _GB, _F), lambda i: (i, 0, 0)),
            pl.BlockSpec(memory_space=pltpu.SMEM),
            pl.BlockSpec(memory_space=pltpu.SMEM),
        ],
        out_specs=pl.BlockSpec((_GB, _N_PV, _OUTC), lambda i: (i, 0, 0)),
        out_shape=jax.ShapeDtypeStruct((_B, _N_PV, _OUTC), jnp.float32),
    )(y, x, emb, t, az, el)


def kernel(pv_y_osgb_fourier, pv_x_osgb_fourier, pv_system_row_number,
           pv_time_utc_fourier, pv_x_osgb, solar_azimuth, solar_elevation,
           pv_embedding):
    idx = pv_system_row_number.reshape(-1).astype(jnp.int32)
    table128 = jnp.pad(pv_embedding, ((0, 0), (0, 128 - _EMB)))
    emb = _sc_gather(idx, table128)
    t = pv_time_utc_fourier[:, 12].reshape(_GRID, _GB, _F)
    az = solar_azimuth[:, 12]
    el = solar_elevation[:, 12]
    return _tc_assemble(pv_y_osgb_fourier, pv_x_osgb_fourier, emb, t, az, el)


# speedup vs baseline: 1.6468x; 1.0211x over previous
import functools

import jax
import jax.numpy as jnp
from jax import lax
from jax.experimental import pallas as pl
from jax.experimental.pallas import tpu as pltpu
from jax.experimental.pallas import tpu_sc as plsc

_B = 256
_N_PV = 1400
_F = 64
_EMB = 32
_OUTC = 2 * _F + _F + _EMB + 2
_NROWS = _B * _N_PV

_NC = 2
_NS = 16
_NW = _NC * _NS
_RPW = _NROWS // _NW
_R = 320
_NCH = _RPW // _R

_GB = 4
_GRID = _B // _GB


def _sc_gather(idx, table128):
    mesh = plsc.VectorSubcoreMesh(core_axis_name="c", subcore_axis_name="s")

    @functools.partial(
        pl.kernel,
        mesh=mesh,
        compiler_params=pltpu.CompilerParams(use_tc_tiling_on_sc=False),
        out_type=jax.ShapeDtypeStruct((_NROWS, 128), jnp.float32),
        scratch_types=[
            pltpu.VMEM((2, _R), jnp.int32),
            pltpu.VMEM((2, _R, 128), jnp.float32),
            pltpu.SemaphoreType.DMA((2,)),
            pltpu.SemaphoreType.DMA((2,)),
            pltpu.SemaphoreType.DMA((2,)),
        ],
    )
    def k(idx_hbm, table_hbm, out_hbm, idx_v, emb_v,
          sem_idx, sem_g, sem_out):
        wid = lax.axis_index("s") * _NC + lax.axis_index("c")
        base = wid * _RPW

        def start_idx(c, p):
            return pltpu.async_copy(idx_hbm.at[pl.ds(base + c * _R, _R)],
                                    idx_v.at[p], sem_idx.at[p])

        ins = {0: start_idx(0, 0)}
        outs = {}
        for c in range(_NCH):
            p = c & 1
            q = 1 - p
            if c + 1 < _NCH:
                ins[q] = start_idx(c + 1, q)
            ins.pop(p).wait()
            if c >= 2:
                outs.pop(p).wait()
            pltpu.async_copy(table_hbm.at[idx_v.at[p]], emb_v.at[p],
                             sem_g.at[p]).wait()
            outs[p] = pltpu.async_copy(
                emb_v.at[p], out_hbm.at[pl.ds(base + c * _R, _R)],
                sem_out.at[p])
        for h in outs.values():
            h.wait()

    return k(idx, table128)


def _fix(v):
    return jnp.nan_to_num(v)


def _unpair(v2):
    v_rep = jnp.repeat(v2, 2, axis=0)
    odd = (lax.broadcasted_iota(jnp.int32, (_N_PV, 1), 0) & 1) == 1
    return jnp.where(odd, v_rep[:, _F:], v_rep[:, :_F])


def _tc_body(y_ref, x_ref, emb_ref, t_ref, az_ref, el_ref, out_ref):
    g = pl.program_id(0)
    for bb in range(_GB):
        y = _fix(_unpair(y_ref[bb]))
        x = _fix(_unpair(x_ref[bb]))
        e = _fix(emb_ref[pl.ds(bb * _N_PV, _N_PV), pl.ds(0, _EMB)])
        t = _fix(t_ref[0, bb])
        az = _fix(az_ref[g * _GB + bb])
        el = _fix(el_ref[g * _GB + bb])
        tb = jnp.broadcast_to(t[None, :], (_N_PV, _F))
        azc = jnp.full((_N_PV, 1), az, jnp.float32)
        elc = jnp.full((_N_PV, 1), el, jnp.float32)
        out_ref[bb] = jnp.concatenate([y, x, tb, e, azc, elc], axis=-1)


def _tc_assemble(y, x, emb, t, az, el):
    return pl.pallas_call(
        _tc_body,
        grid=(_GRID,),
        in_specs=[
            pl.BlockSpec((_GB, _N_PV // 2, 128), lambda i: (i, 0, 0)),
            pl.BlockSpec((_GB, _N_PV // 2, 128), lambda i: (i, 0, 0)),
            pl.BlockSpec((_GB * _N_PV, 128), lambda i: (i, 0)),
            pl.BlockSpec((1, _GB, _F), lambda i: (i, 0, 0)),
            pl.BlockSpec(memory_space=pltpu.SMEM),
            pl.BlockSpec(memory_space=pltpu.SMEM),
        ],
        out_specs=pl.BlockSpec((_GB, _N_PV, _OUTC), lambda i: (i, 0, 0)),
        out_shape=jax.ShapeDtypeStruct((_B, _N_PV, _OUTC), jnp.float32),
    )(y, x, emb, t, az, el)


def kernel(pv_y_osgb_fourier, pv_x_osgb_fourier, pv_system_row_number,
           pv_time_utc_fourier, pv_x_osgb, solar_azimuth, solar_elevation,
           pv_embedding):
    idx = pv_system_row_number.reshape(-1).astype(jnp.int32)
    table128 = jnp.pad(pv_embedding, ((0, 0), (0, 128 - _EMB)))
    emb = _sc_gather(idx, table128)
    t = pv_time_utc_fourier[:, 12].reshape(_GRID, _GB, _F)
    az = solar_azimuth[:, 12]
    el = solar_elevation[:, 12]
    y700 = pv_y_osgb_fourier.reshape(_B, _N_PV // 2, 128)
    x700 = pv_x_osgb_fourier.reshape(_B, _N_PV // 2, 128)
    return _tc_assemble(y700, x700, emb, t, az, el)


# speedup vs baseline: 3.0481x; 1.8509x over previous
import functools

import jax
import jax.numpy as jnp
from jax import lax
from jax.experimental import pallas as pl
from jax.experimental.pallas import tpu as pltpu
from jax.experimental.pallas import tpu_sc as plsc

_B = 256
_N_PV = 1400
_F = 64
_EMB = 32
_OUTC = 2 * _F + _F + _EMB + 2
_NROWS = _B * _N_PV

_NC = 2
_NS = 16
_NW = _NC * _NS
_RPW = _NROWS // _NW
_R = 320
_NCH = _RPW // _R

_PBLK = 8
_GRID = _N_PV // _PBLK


def _sc_gather(idx, table128):
    mesh = plsc.VectorSubcoreMesh(core_axis_name="c", subcore_axis_name="s")

    @functools.partial(
        pl.kernel,
        mesh=mesh,
        compiler_params=pltpu.CompilerParams(use_tc_tiling_on_sc=False),
        out_type=jax.ShapeDtypeStruct((_NROWS, 128), jnp.float32),
        scratch_types=[
            pltpu.VMEM((2, _R), jnp.int32),
            pltpu.VMEM((2, _R, 128), jnp.float32),
            pltpu.SemaphoreType.DMA((2,)),
            pltpu.SemaphoreType.DMA((2,)),
            pltpu.SemaphoreType.DMA((2,)),
        ],
    )
    def k(idx_hbm, table_hbm, out_hbm, idx_v, emb_v,
          sem_idx, sem_g, sem_out):
        wid = lax.axis_index("s") * _NC + lax.axis_index("c")
        base = wid * _RPW

        def start_idx(c, p):
            return pltpu.async_copy(idx_hbm.at[pl.ds(base + c * _R, _R)],
                                    idx_v.at[p], sem_idx.at[p])

        ins = {0: start_idx(0, 0)}
        outs = {}
        for c in range(_NCH):
            p = c & 1
            q = 1 - p
            if c + 1 < _NCH:
                ins[q] = start_idx(c + 1, q)
            ins.pop(p).wait()
            if c >= 2:
                outs.pop(p).wait()
            pltpu.async_copy(table_hbm.at[idx_v.at[p]], emb_v.at[p],
                             sem_g.at[p]).wait()
            outs[p] = pltpu.async_copy(
                emb_v.at[p], out_hbm.at[pl.ds(base + c * _R, _R)],
                sem_out.at[p])
        for h in outs.values():
            h.wait()

    return k(idx, table128)


def _fix(v):
    return jnp.where(v != v, jnp.float32(0.0), v)


def _tc_body(y_ref, x_ref, emb_ref, t_ref, az_ref, el_ref, out_ref):
    t = _fix(t_ref[...])
    az = _fix(az_ref[...])
    el = _fix(el_ref[...])
    for pp in range(_PBLK):
        y = _fix(y_ref[pp])
        x = _fix(x_ref[pp])
        e_rows = _fix(emb_ref[:, pp, pl.ds(0, _EMB)])
        e = jnp.transpose(e_rows, (1, 0))
        out_ref[:, pp, :] = jnp.concatenate(
            [y, x, t, e, az, el], axis=0)


def _tc_assemble(y_t, x_t, emb3, t_t, az_r, el_r):
    return pl.pallas_call(
        _tc_body,
        grid=(_GRID,),
        in_specs=[
            pl.BlockSpec((_PBLK, _F, _B), lambda i: (i, 0, 0)),
            pl.BlockSpec((_PBLK, _F, _B), lambda i: (i, 0, 0)),
            pl.BlockSpec((_B, _PBLK, 128), lambda i: (0, i, 0)),
            pl.BlockSpec((_F, _B), lambda i: (0, 0)),
            pl.BlockSpec((1, _B), lambda i: (0, 0)),
            pl.BlockSpec((1, _B), lambda i: (0, 0)),
        ],
        out_specs=pl.BlockSpec((_OUTC, _PBLK, _B), lambda i: (0, i, 0)),
        out_shape=jax.ShapeDtypeStruct((_OUTC, _N_PV, _B), jnp.float32),
    )(y_t, x_t, emb3, t_t, az_r, el_r)


def kernel(pv_y_osgb_fourier, pv_x_osgb_fourier, pv_system_row_number,
           pv_time_utc_fourier, pv_x_osgb, solar_azimuth, solar_elevation,
           pv_embedding):
    idx = pv_system_row_number.reshape(-1).astype(jnp.int32)
    table128 = jnp.pad(pv_embedding, ((0, 0), (0, 128 - _EMB)))
    emb = _sc_gather(idx, table128)
    emb3 = emb.reshape(_B, _N_PV, 128)
    y_t = jnp.transpose(pv_y_osgb_fourier, (1, 2, 0))
    x_t = jnp.transpose(pv_x_osgb_fourier, (1, 2, 0))
    t_t = jnp.transpose(pv_time_utc_fourier[:, 12], (1, 0))
    az_r = solar_azimuth[:, 12].reshape(1, _B)
    el_r = solar_elevation[:, 12].reshape(1, _B)
    out_t = _tc_assemble(y_t, x_t, emb3, t_t, az_r, el_r)
    return jnp.transpose(out_t, (2, 1, 0))
